# initial kernel scaffold (unmeasured)
import jax
import jax.numpy as jnp
from jax import lax
from jax.experimental import pallas as pl
from jax.experimental.pallas import tpu as pltpu


def kernel(
    x,
):
    def body(*refs):
        pass

    out_shape = jax.ShapeDtypeStruct(..., jnp.float32)
    return pl.pallas_call(body, out_shape=out_shape)(...)


# baseline (device time: 80597 ns/iter reference)
import jax
import jax.numpy as jnp
from jax import lax
from jax.experimental import pallas as pl
from jax.experimental.pallas import tpu as pltpu

N_DEV = 8
BLK = 512


def kernel(x):
    m, n = x.shape

    def body(x_ref, out_ref, send_sems, recv_sems):
        me = lax.axis_index("i")

        rdmas = []
        for d in range(1, N_DEV):
            dst = lax.rem(me + d, N_DEV)
            rdma = pltpu.make_async_remote_copy(
                src_ref=x_ref.at[:, pl.ds(dst * BLK, BLK)],
                dst_ref=out_ref.at[pl.ds(me * BLK, BLK), :],
                send_sem=send_sems.at[d - 1],
                recv_sem=recv_sems.at[d - 1],
                device_id=(dst,),
                device_id_type=pl.DeviceIdType.MESH,
            )
            rdma.start()
            rdmas.append(rdma)

        out_ref[pl.ds(me * BLK, BLK), :] = x_ref[:, pl.ds(me * BLK, BLK)]

        for rdma in rdmas:
            rdma.wait()

    return pl.pallas_call(
        body,
        out_shape=jax.ShapeDtypeStruct((N_DEV * m, n // N_DEV), x.dtype),
        in_specs=[pl.BlockSpec(memory_space=pltpu.VMEM)],
        out_specs=pl.BlockSpec(memory_space=pltpu.VMEM),
        scratch_shapes=[
            pltpu.SemaphoreType.DMA((N_DEV - 1,)),
            pltpu.SemaphoreType.DMA((N_DEV - 1,)),
        ],
    )(x)


# device time: 26778 ns/iter; 3.0098x vs baseline; 3.0098x over previous
import jax
import jax.numpy as jnp
from jax import lax
from jax.experimental import pallas as pl
from jax.experimental.pallas import tpu as pltpu

N_DEV = 8
BLK = 512


def kernel(x):
    m, n = x.shape

    def body(x_ref, out_ref, send_sems, recv_sems):
        me = lax.axis_index("i")

        rdmas = []
        for d in [1, 3, 4]:
            dst = lax.bitwise_xor(me, d)
            rdma = pltpu.make_async_remote_copy(
                src_ref=x_ref.at[:, pl.ds(dst * BLK, BLK)],
                dst_ref=out_ref.at[pl.ds(me * BLK, BLK), :],
                send_sem=send_sems.at[[1, 3, 4].index(d)],
                recv_sem=recv_sems.at[[1, 3, 4].index(d)],
                device_id=(dst,),
                device_id_type=pl.DeviceIdType.MESH,
            )
            rdma.start()
            rdmas.append(rdma)

        out_ref[pl.ds(me * BLK, BLK), :] = x_ref[:, pl.ds(me * BLK, BLK)]

        for rdma in rdmas:
            rdma.wait()

    return pl.pallas_call(
        body,
        out_shape=jax.ShapeDtypeStruct((N_DEV * m, n // N_DEV), x.dtype),
        in_specs=[pl.BlockSpec(memory_space=pltpu.VMEM)],
        out_specs=pl.BlockSpec(memory_space=pltpu.VMEM),
        scratch_shapes=[
            pltpu.SemaphoreType.DMA((N_DEV - 1,)),
            pltpu.SemaphoreType.DMA((N_DEV - 1,)),
        ],
    )(x)
